# baseline (speedup 1.0000x reference)
import functools

import jax
import jax.numpy as jnp
from jax import lax
from jax.experimental import pallas as pl
from jax.experimental.pallas import tpu as pltpu
from jax.experimental.pallas import tpu_sc as plsc

N = 10000
E = 160000
D_IN = 512
H = 2
C = 256

NP = 10240
NTILE = 16
NB = 96
KB = 112
EP = NTILE * NB * KB
ACCR = 10112
RPT = ACCR // NTILE
ROW_BLK = 1024
TRASH = N


def _xp_body(x_ref, w_ref, attsrc_ref, attdst_ref, xps_ref, asrc_ref, adst_ref):
    xb = x_ref[...]
    xp = jnp.dot(xb, w_ref[...], preferred_element_type=jnp.float32)
    for cc in range(4):
        xps_ref[cc] = xp[:, cc * 128:(cc + 1) * 128]
    ws = xp * attsrc_ref[...]
    wd = xp * attdst_ref[...]
    asrc_ref[0, :] = jnp.sum(ws[:, :C], axis=1)
    asrc_ref[1, :] = jnp.sum(ws[:, C:], axis=1)
    adst_ref[0, :] = jnp.sum(wd[:, :C], axis=1)
    adst_ref[1, :] = jnp.sum(wd[:, C:], axis=1)


def _compute_xp(x, W_conv, att_src_flat, att_dst_flat):
    return pl.pallas_call(
        _xp_body,
        grid=(NP // ROW_BLK,),
        in_specs=[
            pl.BlockSpec((ROW_BLK, D_IN), lambda i: (i, 0)),
            pl.BlockSpec((D_IN, H * C), lambda i: (0, 0)),
            pl.BlockSpec((1, H * C), lambda i: (0, 0)),
            pl.BlockSpec((1, H * C), lambda i: (0, 0)),
        ],
        out_specs=[
            pl.BlockSpec((4, ROW_BLK, 128), lambda i: (0, i, 0)),
            pl.BlockSpec((H, ROW_BLK), lambda i: (0, i)),
            pl.BlockSpec((H, ROW_BLK), lambda i: (0, i)),
        ],
        out_shape=[
            jax.ShapeDtypeStruct((4, NP, 128), jnp.float32),
            jax.ShapeDtypeStruct((H, NP), jnp.float32),
            jax.ShapeDtypeStruct((H, NP), jnp.float32),
        ],
    )(x, W_conv, att_src_flat, att_dst_flat)


def _leaky(a):
    return jnp.maximum(a, 0.0) + 0.2 * jnp.minimum(a, 0.0)


def _sc_pass1_body(src_hbm, dst_hbm, asrc_hbm, adst_hbm,
                   w_hbm, den_hbm,
                   src_v, dst_v, w_v, asrc_v, adst_v, den_v):
    c = lax.axis_index("c")
    s = lax.axis_index("s")

    pltpu.sync_copy(src_hbm.at[s], src_v)
    pltpu.sync_copy(dst_hbm.at[s], dst_v)
    pltpu.sync_copy(asrc_hbm.at[c], asrc_v)
    pltpu.sync_copy(adst_hbm.at[c], adst_v)

    zero16 = jnp.zeros((16,), jnp.float32)

    def zden(i, _):
        den_v[pl.ds(i * 16, 16)] = zero16
        return 0
    lax.fori_loop(0, NP // 16, zden, 0)

    def p1(j, _):
        for g in range(KB // 16):
            sl = pl.ds(g * 16, 16)
            s16 = src_v[j, sl]
            d16 = dst_v[j, sl]
            a = plsc.load_gather(asrc_v, [s16]) + plsc.load_gather(adst_v, [d16])
            w = jnp.exp(_leaky(a))
            w_v[j, sl] = w
            plsc.addupdate_scatter(den_v, [d16], w)
        return 0
    lax.fori_loop(0, NB, p1, 0)

    pltpu.sync_copy(w_v, w_hbm.at[c, s])
    pltpu.sync_copy(den_v, den_hbm.at[c, s])


def _sc_pass2_body(src_hbm, dst_hbm, w_hbm, xps_hbm,
                   msg_hbm,
                   idxr, dstr, wb, rows0, rows1, rows2, acc,
                   ssem0, ssem1, ssem2, ssem3,
                   gsem0, gsem1, gsem2, csem0, csem1, csem2):
    c = lax.axis_index("c")
    s = lax.axis_index("s")

    zero16 = jnp.zeros((16,), jnp.float32)
    ssems = (ssem0, ssem1, ssem2, ssem3)
    gsems = (gsem0, gsem1, gsem2)
    csems = (csem0, csem1, csem2)
    rows = (rows0, rows1, rows2)

    def stage(j, r):
        pltpu.async_copy(src_hbm.at[s, j], idxr.at[r], ssems[r])
        pltpu.async_copy(dst_hbm.at[s, j], dstr.at[r], ssems[r])
        pltpu.async_copy(w_hbm.at[c, s, j], wb.at[r], ssems[r])

    def wait_stage(j, r):
        pltpu.make_async_copy(src_hbm.at[s, j], idxr.at[r], ssems[r]).wait()
        pltpu.make_async_copy(dst_hbm.at[s, j], dstr.at[r], ssems[r]).wait()
        pltpu.make_async_copy(w_hbm.at[c, s, j], wb.at[r], ssems[r]).wait()

    def addoff(r, delta):
        for g in range(KB // 16):
            sl = pl.ds(g * 16, 16)
            idxr[r, sl] = idxr[r, sl] + delta

    def gather(r3, r4):
        pltpu.async_copy(xps_hbm.at[idxr.at[r4]], rows[r3], gsems[r3])

    def wait_gather(r3, r4):
        pltpu.make_async_copy(xps_hbm.at[idxr.at[r4]], rows[r3],
                              gsems[r3]).wait()

    def scatter(r3, r4):
        pltpu.async_copy(rows[r3], acc.at[dstr.at[r4]], csems[r3], add=True)

    def wait_scatter(r3, r4):
        pltpu.make_async_copy(rows[r3], acc.at[dstr.at[r4]],
                              csems[r3]).wait()

    def scale(r3, r4):
        buf = rows[r3]

        def srow(row, _):
            base = pl.multiple_of(row & ~15, 16)
            lane = row & 15
            wv = wb[r4, pl.ds(base, 16)]
            wsplat = wv.at[lax.broadcast(lane, (16,))].get(
                mode="promise_in_bounds")
            for g in range(8):
                sl = pl.ds(g * 16, 16)
                buf[row, sl] = buf[row, sl] * wsplat
            return 0
        lax.fori_loop(0, KB, srow, 0)

    def run_chunk(cc, delta):
        def zrows(row, _):
            for g in range(8):
                rows0[row, pl.ds(g * 16, 16)] = zero16
            return 0
        lax.fori_loop(0, KB, zrows, 0)
        nfull, rem = RPT // KB, RPT % KB
        for k in range(nfull):
            pltpu.sync_copy(rows0, acc.at[pl.ds(s * RPT + k * KB, KB)])
        if rem:
            pltpu.sync_copy(rows0.at[pl.ds(0, rem)],
                            acc.at[pl.ds(s * RPT + nfull * KB, rem)])
        plsc.subcore_barrier()

        chunk = c * 2 + cc

        stage(0, 0)
        stage(1, 1)
        wait_stage(0, 0)
        addoff(0, delta)
        gather(0, 0)

        def step(j, t):
            r3, r4 = t % 3, t % 4

            @pl.when(j >= 2)
            def _():
                wait_scatter((t - 2) % 3, (t - 2) % 4)

            @pl.when(j + 2 < NB)
            def _():
                stage(j + 2, (t + 2) % 4)

            @pl.when(j + 1 < NB)
            def _():
                wait_stage(j + 1, (t + 1) % 4)
                addoff((t + 1) % 4, delta)
                gather((t + 1) % 3, (t + 1) % 4)

            wait_gather(r3, r4)
            scale(r3, r4)
            scatter(r3, r4)

        def body(i, _):
            for t in range(12):
                step(i * 12 + t, t)
            return 0
        lax.fori_loop(0, NB // 12, body, 0)
        wait_scatter((NB - 2) % 3, (NB - 2) % 4)
        wait_scatter((NB - 1) % 3, (NB - 1) % 4)
        plsc.subcore_barrier()

        pltpu.sync_copy(acc.at[pl.ds(s * RPT, RPT)],
                        msg_hbm.at[chunk, pl.ds(s * RPT, RPT)])
        plsc.subcore_barrier()

    run_chunk(0, (c * 2) * NP)
    run_chunk(1, (c * 2 + 1) * NP)


def _sc_edge(src3, dst3, asrc, adst, xps):
    mesh = plsc.VectorSubcoreMesh(core_axis_name="c", subcore_axis_name="s",
                                  num_cores=2, num_subcores=NTILE)
    p1 = pl.kernel(
        _sc_pass1_body,
        compiler_params=pltpu.CompilerParams(needs_layout_passes=False),
        out_type=[
            jax.ShapeDtypeStruct((H, NTILE, NB, KB), jnp.float32),
            jax.ShapeDtypeStruct((H, NTILE, NP), jnp.float32),
        ],
        mesh=mesh,
        scratch_types=[
            pltpu.VMEM((NB, KB), jnp.int32),
            pltpu.VMEM((NB, KB), jnp.int32),
            pltpu.VMEM((NB, KB), jnp.float32),
            pltpu.VMEM((NP,), jnp.float32),
            pltpu.VMEM((NP,), jnp.float32),
            pltpu.VMEM((NP,), jnp.float32),
        ],
    )
    w, den = p1(src3, dst3, asrc, adst)

    p2 = pl.kernel(
        _sc_pass2_body,
        compiler_params=pltpu.CompilerParams(needs_layout_passes=False),
        out_type=[
            jax.ShapeDtypeStruct((4, NP, 128), jnp.float32),
        ],
        mesh=mesh,
        scratch_types=[
            pltpu.VMEM((4, KB), jnp.int32),
            pltpu.VMEM((4, KB), jnp.int32),
            pltpu.VMEM((4, KB), jnp.float32),
            pltpu.VMEM((KB, 128), jnp.float32),
            pltpu.VMEM((KB, 128), jnp.float32),
            pltpu.VMEM((KB, 128), jnp.float32),
            pltpu.VMEM_SHARED((ACCR, 128), jnp.float32),
        ] + [pltpu.SemaphoreType.DMA] * 10,
    )
    (msg,) = p2(src3, dst3, w, xps.reshape(4 * NP, 128))
    return msg, den


def _mlp_body(msg0_ref, msg1_ref, msg2_ref, msg3_ref, den_ref,
              asrc_ref, adst_ref, xp0_ref, xp1_ref, xp2_ref, xp3_ref,
              bconv_ref, wa_ref, ba_ref, ga_ref, bta_ref,
              w1_ref, b1_ref, g1_ref, bt1_ref,
              w2_ref, b2_ref, g2_ref, bt2_ref,
              w3_ref, b3_ref, p_ref):
    def ln(v, g, b):
        mu = jnp.mean(v, axis=-1, keepdims=True)
        var = jnp.mean((v - mu) ** 2, axis=-1, keepdims=True)
        return (v - mu) * lax.rsqrt(var + 1e-5) * g + b

    wself = jnp.exp(_leaky(asrc_ref[...] + adst_ref[...]))
    den = jnp.sum(den_ref[...], axis=1) + wself + 1e-16
    inv0 = (1.0 / den[0])[:, None]
    inv1 = (1.0 / den[1])[:, None]
    ws0 = wself[0][:, None]
    ws1 = wself[1][:, None]
    h0 = jnp.concatenate([msg0_ref[0], msg1_ref[0]], axis=1)
    h1 = jnp.concatenate([msg2_ref[0], msg3_ref[0]], axis=1)
    xp0 = jnp.concatenate([xp0_ref[0], xp1_ref[0]], axis=1)
    xp1 = jnp.concatenate([xp2_ref[0], xp3_ref[0]], axis=1)
    h = jnp.concatenate([(h0 + ws0 * xp0) * inv0,
                         (h1 + ws1 * xp1) * inv1], axis=1)
    h = jnp.maximum(h + bconv_ref[...], 0.0)
    h = jnp.dot(h, wa_ref[...], preferred_element_type=jnp.float32) + ba_ref[...]
    h = ln(h, ga_ref[...], bta_ref[...])
    h = jnp.maximum(h, 0.0)
    h = jnp.dot(h, w1_ref[...], preferred_element_type=jnp.float32) + b1_ref[...]
    h = ln(h, g1_ref[...], bt1_ref[...])
    h = jnp.tanh(jnp.maximum(h, 0.0))
    h = jnp.dot(h, w2_ref[...], preferred_element_type=jnp.float32) + b2_ref[...]
    h = ln(h, g2_ref[...], bt2_ref[...])
    h = jnp.maximum(h, 0.0)
    p_ref[...] = jnp.dot(h, w3_ref[...], preferred_element_type=jnp.float32) + b3_ref[...]


def _mlp(msg, den, asrc, adst, xps, b_conv, Wa, ba, ga, bta,
         W1, b1, g1, bt1, W2, b2, g2, bt2, W3, b3):
    full = lambda r, c: pl.BlockSpec((r, c), lambda i: (0, 0))
    row = lambda c: pl.BlockSpec((1, c), lambda i: (0, 0))
    chunk = lambda cc: pl.BlockSpec((1, ROW_BLK, 128), lambda i, cc=cc: (cc, i, 0))
    hblk = pl.BlockSpec((H, ROW_BLK), lambda i: (0, i))
    return pl.pallas_call(
        _mlp_body,
        grid=(NP // ROW_BLK,),
        in_specs=[
            chunk(0), chunk(1), chunk(2), chunk(3),
            pl.BlockSpec((H, NTILE, ROW_BLK), lambda i: (0, 0, i)),
            hblk, hblk,
            chunk(0), chunk(1), chunk(2), chunk(3),
            row(H * C), full(H * C, 256), row(256), row(256), row(256),
            full(256, 128), row(128), row(128), row(128),
            full(128, 64), row(64), row(64), row(64),
            full(64, 3), row(3),
        ],
        out_specs=pl.BlockSpec((ROW_BLK, 3), lambda i: (i, 0)),
        out_shape=jax.ShapeDtypeStruct((N, 3), jnp.float32),
    )(msg, msg, msg, msg, den, asrc, adst, xps, xps, xps, xps,
      b_conv.reshape(1, -1), Wa, ba.reshape(1, -1), ga.reshape(1, -1),
      bta.reshape(1, -1), W1, b1.reshape(1, -1), g1.reshape(1, -1),
      bt1.reshape(1, -1), W2, b2.reshape(1, -1), g2.reshape(1, -1),
      bt2.reshape(1, -1), W3, b3.reshape(1, -1))


CD_RB = 1024
CD_CB = 2048


def _cdist_body(pi_ref, pj_ref, out_ref):
    pi = pi_ref[...]
    pj = pj_ref[...]
    dots = lax.dot_general(pi, pj, (((1,), (1,)), ((), ())),
                           preferred_element_type=jnp.float32)
    sq_i = jnp.sum(pi * pi, axis=1, keepdims=True)
    sq_j = jnp.sum(pj * pj, axis=1, keepdims=True)
    d2 = sq_i + jnp.transpose(sq_j) - 2.0 * dots
    d2 = jnp.maximum(d2, 0.0)
    out_ref[...] = jnp.where(d2 > 0.0, jnp.sqrt(jnp.where(d2 > 0.0, d2, 1.0)), 0.0)


def _cdist(p):
    grid = (pl.cdiv(N, CD_RB), pl.cdiv(N, CD_CB))
    return pl.pallas_call(
        _cdist_body,
        grid=grid,
        in_specs=[
            pl.BlockSpec((CD_RB, 3), lambda i, j: (i, 0)),
            pl.BlockSpec((CD_CB, 3), lambda i, j: (j, 0)),
        ],
        out_specs=pl.BlockSpec((CD_RB, CD_CB), lambda i, j: (i, j)),
        out_shape=jax.ShapeDtypeStruct((N, N), jnp.float32),
    )(p, p)


def kernel(x, edge_index, W_conv, att_src, att_dst, b_conv, Wa, ba, ga, bta,
           W1, b1, g1, bt1, W2, b2, g2, bt2, W3, b3):
    xps, asrc, adst = _compute_xp(x, W_conv, att_src.reshape(1, H * C),
                                  att_dst.reshape(1, H * C))
    pad = EP - E
    src3 = jnp.concatenate(
        [edge_index[0], jnp.zeros((pad,), jnp.int32)]).reshape(NTILE, NB, KB)
    dst3 = jnp.concatenate(
        [edge_index[1], jnp.full((pad,), TRASH, jnp.int32)]).reshape(NTILE, NB, KB)
    msg, den = _sc_edge(src3, dst3, asrc, adst, xps)
    p = _mlp(msg, den, asrc, adst, xps, b_conv, Wa, ba, ga, bta,
             W1, b1, g1, bt1, W2, b2, g2, bt2, W3, b3)
    return _cdist(p)


# speedup vs baseline: 1.2443x; 1.2443x over previous
import functools

import jax
import jax.numpy as jnp
from jax import lax
from jax.experimental import pallas as pl
from jax.experimental.pallas import tpu as pltpu
from jax.experimental.pallas import tpu_sc as plsc

N = 10000
E = 160000
D_IN = 512
H = 2
C = 256

NP = 10240
NTILE = 16
NB = 128
KB = 80
EP = NTILE * NB * KB
ACCR = 10112
RPT = ACCR // NTILE
ROW_BLK = 1024
TRASH = N


def _xp_body(x_ref, w_ref, attsrc_ref, attdst_ref, xps_ref, asrc_ref, adst_ref):
    xb = x_ref[...]
    xp = jnp.dot(xb, w_ref[...], preferred_element_type=jnp.float32)
    for cc in range(4):
        xps_ref[cc] = xp[:, cc * 128:(cc + 1) * 128]
    ws = xp * attsrc_ref[...]
    wd = xp * attdst_ref[...]
    asrc_ref[0, :] = jnp.sum(ws[:, :C], axis=1)
    asrc_ref[1, :] = jnp.sum(ws[:, C:], axis=1)
    adst_ref[0, :] = jnp.sum(wd[:, :C], axis=1)
    adst_ref[1, :] = jnp.sum(wd[:, C:], axis=1)


def _compute_xp(x, W_conv, att_src_flat, att_dst_flat):
    return pl.pallas_call(
        _xp_body,
        grid=(NP // ROW_BLK,),
        in_specs=[
            pl.BlockSpec((ROW_BLK, D_IN), lambda i: (i, 0)),
            pl.BlockSpec((D_IN, H * C), lambda i: (0, 0)),
            pl.BlockSpec((1, H * C), lambda i: (0, 0)),
            pl.BlockSpec((1, H * C), lambda i: (0, 0)),
        ],
        out_specs=[
            pl.BlockSpec((4, ROW_BLK, 128), lambda i: (0, i, 0)),
            pl.BlockSpec((H, ROW_BLK), lambda i: (0, i)),
            pl.BlockSpec((H, ROW_BLK), lambda i: (0, i)),
        ],
        out_shape=[
            jax.ShapeDtypeStruct((4, NP, 128), jnp.float32),
            jax.ShapeDtypeStruct((H, NP), jnp.float32),
            jax.ShapeDtypeStruct((H, NP), jnp.float32),
        ],
    )(x, W_conv, att_src_flat, att_dst_flat)


def _leaky(a):
    return jnp.maximum(a, 0.0) + 0.2 * jnp.minimum(a, 0.0)


def _sc_pass1_body(src_hbm, dst_hbm, asrc_hbm, adst_hbm,
                   w_hbm, den_hbm,
                   src_v, dst_v, w_v, asrc_v, adst_v, den_v):
    c = lax.axis_index("c")
    s = lax.axis_index("s")

    pltpu.sync_copy(src_hbm.at[s], src_v)
    pltpu.sync_copy(dst_hbm.at[s], dst_v)
    pltpu.sync_copy(asrc_hbm.at[c], asrc_v)
    pltpu.sync_copy(adst_hbm.at[c], adst_v)

    zero16 = jnp.zeros((16,), jnp.float32)

    def zden(i, _):
        den_v[pl.ds(i * 16, 16)] = zero16
        return 0
    lax.fori_loop(0, NP // 16, zden, 0)

    def p1(j, _):
        for g in range(KB // 16):
            sl = pl.ds(g * 16, 16)
            s16 = src_v[j, sl]
            d16 = dst_v[j, sl]
            a = plsc.load_gather(asrc_v, [s16]) + plsc.load_gather(adst_v, [d16])
            w = jnp.exp(_leaky(a))
            w_v[j, sl] = w
            plsc.addupdate_scatter(den_v, [d16], w)
        return 0
    lax.fori_loop(0, NB, p1, 0)

    pltpu.sync_copy(w_v, w_hbm.at[c, s])
    pltpu.sync_copy(den_v, den_hbm.at[c, s])


def _sc_pass2_body(src_hbm, dst_hbm, w_hbm, xps_hbm,
                   msg_hbm,
                   sidx_v, dstb, wb, rows0, rows1, acc,
                   gsem, csem0, csem1):
    c = lax.axis_index("c")
    s = lax.axis_index("s")

    pltpu.sync_copy(src_hbm.at[s], sidx_v)

    zero16 = jnp.zeros((16,), jnp.float32)
    csems = (csem0, csem1)
    rows = (rows0, rows1)

    def scatter(r2):
        pltpu.async_copy(rows[r2], acc.at[dstb.at[r2]], csems[r2], add=True)

    def wait_scatter(r2):
        pltpu.make_async_copy(rows[r2], acc.at[dstb.at[r2]],
                              csems[r2]).wait()

    def scale(r2):
        buf = rows[r2]

        def srow(row, _):
            base = pl.multiple_of(row & ~15, 16)
            lane = row & 15
            wv = wb[r2, pl.ds(base, 16)]
            wsplat = wv.at[lax.broadcast(lane, (16,))].get(
                mode="promise_in_bounds")
            for g in range(8):
                sl = pl.ds(g * 16, 16)
                buf[row, sl] = buf[row, sl] * wsplat
            return 0
        lax.fori_loop(0, KB, srow, 0)

    def run_chunk(cc, delta):
        def zrows(row, _):
            for g in range(8):
                rows0[row, pl.ds(g * 16, 16)] = zero16
            return 0
        lax.fori_loop(0, KB, zrows, 0)
        nfull, rem = RPT // KB, RPT % KB
        for k in range(nfull):
            pltpu.sync_copy(rows0, acc.at[pl.ds(s * RPT + k * KB, KB)])
        if rem:
            pltpu.sync_copy(rows0.at[pl.ds(0, rem)],
                            acc.at[pl.ds(s * RPT + nfull * KB, rem)])
        plsc.subcore_barrier()

        chunk = c * 2 + cc

        def mkidx(j, _):
            for g in range(KB // 16):
                sl = pl.ds(g * 16, 16)
                sidx_v[j, sl] = sidx_v[j, sl] + delta
            return 0
        lax.fori_loop(0, NB, mkidx, 0)

        def body(i, _):
            for b in range(2):
                j = i * 2 + b

                @pl.when(j >= 2)
                def _():
                    wait_scatter(b)

                pltpu.sync_copy(dst_hbm.at[s, j], dstb.at[b])
                pltpu.sync_copy(w_hbm.at[c, s, j], wb.at[b])
                pltpu.async_copy(xps_hbm.at[sidx_v.at[j]], rows[b],
                                 gsem).wait()
                scale(b)
                scatter(b)
            return 0
        lax.fori_loop(0, NB // 2, body, 0)
        wait_scatter(0)
        wait_scatter(1)
        plsc.subcore_barrier()

        pltpu.sync_copy(acc.at[pl.ds(s * RPT, RPT)],
                        msg_hbm.at[chunk, pl.ds(s * RPT, RPT)])
        plsc.subcore_barrier()

    run_chunk(0, (c * 2) * NP)
    run_chunk(1, NP)


def _sc_edge(src3, dst3, asrc, adst, xps):
    mesh = plsc.VectorSubcoreMesh(core_axis_name="c", subcore_axis_name="s",
                                  num_cores=2, num_subcores=NTILE)
    p1 = pl.kernel(
        _sc_pass1_body,
        compiler_params=pltpu.CompilerParams(needs_layout_passes=False),
        out_type=[
            jax.ShapeDtypeStruct((H, NTILE, NB, KB), jnp.float32),
            jax.ShapeDtypeStruct((H, NTILE, NP), jnp.float32),
        ],
        mesh=mesh,
        scratch_types=[
            pltpu.VMEM((NB, KB), jnp.int32),
            pltpu.VMEM((NB, KB), jnp.int32),
            pltpu.VMEM((NB, KB), jnp.float32),
            pltpu.VMEM((NP,), jnp.float32),
            pltpu.VMEM((NP,), jnp.float32),
            pltpu.VMEM((NP,), jnp.float32),
        ],
    )
    w, den = p1(src3, dst3, asrc, adst)

    p2 = pl.kernel(
        _sc_pass2_body,
        compiler_params=pltpu.CompilerParams(needs_layout_passes=False),
        out_type=[
            jax.ShapeDtypeStruct((4, NP, 128), jnp.float32),
        ],
        mesh=mesh,
        scratch_types=[
            pltpu.VMEM((NB, KB), jnp.int32),
            pltpu.VMEM((2, KB), jnp.int32),
            pltpu.VMEM((2, KB), jnp.float32),
            pltpu.VMEM((KB, 128), jnp.float32),
            pltpu.VMEM((KB, 128), jnp.float32),
            pltpu.VMEM_SHARED((ACCR, 128), jnp.float32),
        ] + [pltpu.SemaphoreType.DMA] * 3,
    )
    (msg,) = p2(src3, dst3, w, xps.reshape(4 * NP, 128))
    return msg, den


def _mlp_body(msg0_ref, msg1_ref, msg2_ref, msg3_ref, den_ref,
              asrc_ref, adst_ref, xp0_ref, xp1_ref, xp2_ref, xp3_ref,
              bconv_ref, wa_ref, ba_ref, ga_ref, bta_ref,
              w1_ref, b1_ref, g1_ref, bt1_ref,
              w2_ref, b2_ref, g2_ref, bt2_ref,
              w3_ref, b3_ref, p_ref, sq_ref):
    def ln(v, g, b):
        mu = jnp.mean(v, axis=-1, keepdims=True)
        var = jnp.mean((v - mu) ** 2, axis=-1, keepdims=True)
        return (v - mu) * lax.rsqrt(var + 1e-5) * g + b

    wself = jnp.exp(_leaky(asrc_ref[...] + adst_ref[...]))
    den = jnp.sum(den_ref[...], axis=1) + wself + 1e-16
    inv0 = (1.0 / den[0])[:, None]
    inv1 = (1.0 / den[1])[:, None]
    ws0 = wself[0][:, None]
    ws1 = wself[1][:, None]
    h0 = jnp.concatenate([msg0_ref[0], msg1_ref[0]], axis=1)
    h1 = jnp.concatenate([msg2_ref[0], msg3_ref[0]], axis=1)
    xp0 = jnp.concatenate([xp0_ref[0], xp1_ref[0]], axis=1)
    xp1 = jnp.concatenate([xp2_ref[0], xp3_ref[0]], axis=1)
    h = jnp.concatenate([(h0 + ws0 * xp0) * inv0,
                         (h1 + ws1 * xp1) * inv1], axis=1)
    h = jnp.maximum(h + bconv_ref[...], 0.0)
    h = jnp.dot(h, wa_ref[...], preferred_element_type=jnp.float32) + ba_ref[...]
    h = ln(h, ga_ref[...], bta_ref[...])
    h = jnp.maximum(h, 0.0)
    h = jnp.dot(h, w1_ref[...], preferred_element_type=jnp.float32) + b1_ref[...]
    h = ln(h, g1_ref[...], bt1_ref[...])
    h = jnp.tanh(jnp.maximum(h, 0.0))
    h = jnp.dot(h, w2_ref[...], preferred_element_type=jnp.float32) + b2_ref[...]
    h = ln(h, g2_ref[...], bt2_ref[...])
    h = jnp.maximum(h, 0.0)
    p = jnp.dot(h, w3_ref[...], preferred_element_type=jnp.float32) + b3_ref[...]
    p_ref[...] = p
    sq_ref[0, :] = jnp.sum(p * p, axis=1)


def _mlp(msg, den, asrc, adst, xps, b_conv, Wa, ba, ga, bta,
         W1, b1, g1, bt1, W2, b2, g2, bt2, W3, b3):
    full = lambda r, c: pl.BlockSpec((r, c), lambda i: (0, 0))
    row = lambda c: pl.BlockSpec((1, c), lambda i: (0, 0))
    chunk = lambda cc: pl.BlockSpec((1, ROW_BLK, 128), lambda i, cc=cc: (cc, i, 0))
    hblk = pl.BlockSpec((H, ROW_BLK), lambda i: (0, i))
    return pl.pallas_call(
        _mlp_body,
        grid=(NP // ROW_BLK,),
        in_specs=[
            chunk(0), chunk(1), chunk(2), chunk(3),
            pl.BlockSpec((H, NTILE, ROW_BLK), lambda i: (0, 0, i)),
            hblk, hblk,
            chunk(0), chunk(1), chunk(2), chunk(3),
            row(H * C), full(H * C, 256), row(256), row(256), row(256),
            full(256, 128), row(128), row(128), row(128),
            full(128, 64), row(64), row(64), row(64),
            full(64, 3), row(3),
        ],
        out_specs=[pl.BlockSpec((ROW_BLK, 3), lambda i: (i, 0)),
                   pl.BlockSpec((1, ROW_BLK), lambda i: (0, i))],
        out_shape=[jax.ShapeDtypeStruct((N, 3), jnp.float32),
                   jax.ShapeDtypeStruct((1, NP), jnp.float32)],
    )(msg, msg, msg, msg, den, asrc, adst, xps, xps, xps, xps,
      b_conv.reshape(1, -1), Wa, ba.reshape(1, -1), ga.reshape(1, -1),
      bta.reshape(1, -1), W1, b1.reshape(1, -1), g1.reshape(1, -1),
      bt1.reshape(1, -1), W2, b2.reshape(1, -1), g2.reshape(1, -1),
      bt2.reshape(1, -1), W3, b3.reshape(1, -1))


CD_RB = 1024
CD_CB = 2048


def _cdist_body(pi_ref, pj_ref, sqj_ref, out_ref):
    pi = pi_ref[...]
    pj = pj_ref[...]
    dots = lax.dot_general(pi, pj, (((1,), (1,)), ((), ())),
                           preferred_element_type=jnp.float32)
    sq_i = jnp.sum(pi * pi, axis=1, keepdims=True)
    d2 = (sq_i + sqj_ref[...]) - 2.0 * dots
    out_ref[...] = jnp.sqrt(jnp.maximum(d2, 0.0))


def _cdist(p, sq):
    grid = (pl.cdiv(N, CD_RB), pl.cdiv(N, CD_CB))
    return pl.pallas_call(
        _cdist_body,
        grid=grid,
        in_specs=[
            pl.BlockSpec((CD_RB, 3), lambda i, j: (i, 0)),
            pl.BlockSpec((CD_CB, 3), lambda i, j: (j, 0)),
            pl.BlockSpec((1, CD_CB), lambda i, j: (0, j)),
        ],
        out_specs=pl.BlockSpec((CD_RB, CD_CB), lambda i, j: (i, j)),
        out_shape=jax.ShapeDtypeStruct((N, N), jnp.float32),
    )(p, p, sq)


def kernel(x, edge_index, W_conv, att_src, att_dst, b_conv, Wa, ba, ga, bta,
           W1, b1, g1, bt1, W2, b2, g2, bt2, W3, b3):
    xps, asrc, adst = _compute_xp(x, W_conv, att_src.reshape(1, H * C),
                                  att_dst.reshape(1, H * C))
    pad = EP - E
    src3 = jnp.concatenate(
        [edge_index[0], jnp.zeros((pad,), jnp.int32)]).reshape(NTILE, NB, KB)
    dst3 = jnp.concatenate(
        [edge_index[1], jnp.full((pad,), TRASH, jnp.int32)]).reshape(NTILE, NB, KB)
    msg, den = _sc_edge(src3, dst3, asrc, adst, xps)
    p, sq = _mlp(msg, den, asrc, adst, xps, b_conv, Wa, ba, ga, bta,
                 W1, b1, g1, bt1, W2, b2, g2, bt2, W3, b3)
    return _cdist(p, sq)


# speedup vs baseline: 1.6887x; 1.3572x over previous
import functools

import jax
import jax.numpy as jnp
from jax import lax
from jax.experimental import pallas as pl
from jax.experimental.pallas import tpu as pltpu
from jax.experimental.pallas import tpu_sc as plsc

N = 10000
E = 160000
D_IN = 512
H = 2
C = 256

NP = 10240
NTILE = 16
NB = 129
KB = 80
EP = NTILE * NB * KB
ACCR = 10112
RPT = ACCR // NTILE
ROW_BLK = 1024
TRASH = N


def _xp_body(x_ref, w_ref, attsrc_ref, attdst_ref, xps_ref, asrc_ref, adst_ref):
    xb = x_ref[...]
    xp = jnp.dot(xb, w_ref[...], preferred_element_type=jnp.float32)
    for cc in range(4):
        xps_ref[cc] = xp[:, cc * 128:(cc + 1) * 128]
    ws = xp * attsrc_ref[...]
    wd = xp * attdst_ref[...]
    asrc_ref[0, :] = jnp.sum(ws[:, :C], axis=1)
    asrc_ref[1, :] = jnp.sum(ws[:, C:], axis=1)
    adst_ref[0, :] = jnp.sum(wd[:, :C], axis=1)
    adst_ref[1, :] = jnp.sum(wd[:, C:], axis=1)


def _compute_xp(x, W_conv, att_src_flat, att_dst_flat):
    return pl.pallas_call(
        _xp_body,
        grid=(NP // ROW_BLK,),
        in_specs=[
            pl.BlockSpec((ROW_BLK, D_IN), lambda i: (i, 0)),
            pl.BlockSpec((D_IN, H * C), lambda i: (0, 0)),
            pl.BlockSpec((1, H * C), lambda i: (0, 0)),
            pl.BlockSpec((1, H * C), lambda i: (0, 0)),
        ],
        out_specs=[
            pl.BlockSpec((4, ROW_BLK, 128), lambda i: (0, i, 0)),
            pl.BlockSpec((H, ROW_BLK), lambda i: (0, i)),
            pl.BlockSpec((H, ROW_BLK), lambda i: (0, i)),
        ],
        out_shape=[
            jax.ShapeDtypeStruct((4, NP, 128), jnp.float32),
            jax.ShapeDtypeStruct((H, NP), jnp.float32),
            jax.ShapeDtypeStruct((H, NP), jnp.float32),
        ],
    )(x, W_conv, att_src_flat, att_dst_flat)


def _leaky(a):
    return jnp.maximum(a, 0.0) + 0.2 * jnp.minimum(a, 0.0)


def _sc_pass1_body(src_hbm, dst_hbm, asrc_hbm, adst_hbm,
                   w_hbm, den_hbm,
                   src_v, dst_v, w_v, asrc_v, adst_v, den_v):
    c = lax.axis_index("c")
    s = lax.axis_index("s")

    pltpu.sync_copy(src_hbm.at[s], src_v)
    pltpu.sync_copy(dst_hbm.at[s], dst_v)
    pltpu.sync_copy(asrc_hbm.at[c], asrc_v)
    pltpu.sync_copy(adst_hbm.at[c], adst_v)

    zero16 = jnp.zeros((16,), jnp.float32)

    def zden(i, _):
        den_v[pl.ds(i * 16, 16)] = zero16
        return 0
    lax.fori_loop(0, NP // 16, zden, 0)

    def p1(j, _):
        for g in range(KB // 16):
            sl = pl.ds(g * 16, 16)
            s16 = src_v[j, sl]
            d16 = dst_v[j, sl]
            a = plsc.load_gather(asrc_v, [s16]) + plsc.load_gather(adst_v, [d16])
            w = jnp.exp(_leaky(a))
            w_v[j, sl] = w
            plsc.addupdate_scatter(den_v, [d16], w)
        return 0
    lax.fori_loop(0, NB, p1, 0)

    pltpu.sync_copy(w_v, w_hbm.at[c, s])
    pltpu.sync_copy(den_v, den_hbm.at[c, s])


def _sc_pass2_body(src_hbm, dst_hbm, w_hbm, xps_hbm,
                   msg_hbm,
                   sidx_v, dstb, wb, rows0, rows1, rows2, acc,
                   ssem0, ssem1, ssem2, gsem0, gsem1, gsem2,
                   csem0, csem1, csem2):
    c = lax.axis_index("c")
    s = lax.axis_index("s")

    pltpu.sync_copy(src_hbm.at[s], sidx_v)

    zero16 = jnp.zeros((16,), jnp.float32)
    ssems = (ssem0, ssem1, ssem2)
    gsems = (gsem0, gsem1, gsem2)
    csems = (csem0, csem1, csem2)
    rows = (rows0, rows1, rows2)

    def gather(j, r):
        pltpu.async_copy(xps_hbm.at[sidx_v.at[j]], rows[r], gsems[r])

    def wait_gather(j, r):
        pltpu.make_async_copy(xps_hbm.at[sidx_v.at[j]], rows[r],
                              gsems[r]).wait()

    def stage(j, r):
        pltpu.async_copy(dst_hbm.at[s, j], dstb.at[r], ssems[r])
        pltpu.async_copy(w_hbm.at[c, s, j], wb.at[r], ssems[r])

    def wait_stage(j, r):
        pltpu.make_async_copy(dst_hbm.at[s, j], dstb.at[r], ssems[r]).wait()
        pltpu.make_async_copy(w_hbm.at[c, s, j], wb.at[r], ssems[r]).wait()

    def scatter(r2):
        pltpu.async_copy(rows[r2], acc.at[dstb.at[r2]], csems[r2], add=True)

    def wait_scatter(r2):
        pltpu.make_async_copy(rows[r2], acc.at[dstb.at[r2]],
                              csems[r2]).wait()

    def scale(r2):
        buf = rows[r2]

        def srow(row, _):
            base = pl.multiple_of(row & ~15, 16)
            lane = row & 15
            wv = wb[r2, pl.ds(base, 16)]
            wsplat = wv.at[lax.broadcast(lane, (16,))].get(
                mode="promise_in_bounds")
            for g in range(8):
                sl = pl.ds(g * 16, 16)
                buf[row, sl] = buf[row, sl] * wsplat
            return 0
        lax.fori_loop(0, KB, srow, 0)

    def run_chunk(cc, delta):
        def zrows(row, _):
            for g in range(8):
                rows0[row, pl.ds(g * 16, 16)] = zero16
            return 0
        lax.fori_loop(0, KB, zrows, 0)
        nfull, rem = RPT // KB, RPT % KB
        for k in range(nfull):
            pltpu.sync_copy(rows0, acc.at[pl.ds(s * RPT + k * KB, KB)])
        if rem:
            pltpu.sync_copy(rows0.at[pl.ds(0, rem)],
                            acc.at[pl.ds(s * RPT + nfull * KB, rem)])
        plsc.subcore_barrier()

        chunk = c * 2 + cc

        def mkidx(j, _):
            for g in range(KB // 16):
                sl = pl.ds(g * 16, 16)
                sidx_v[j, sl] = sidx_v[j, sl] + delta
            return 0
        lax.fori_loop(0, NB, mkidx, 0)

        gather(0, 0)
        stage(0, 0)

        def body(i, _):
            for t in range(3):
                j = i * 3 + t

                @pl.when(j >= 2)
                def _():
                    wait_scatter((t + 1) % 3)

                @pl.when(j + 1 < NB)
                def _():
                    gather(j + 1, (t + 1) % 3)
                    stage(j + 1, (t + 1) % 3)

                wait_gather(j, t)
                wait_stage(j, t)
                scale(t)
                scatter(t)
            return 0
        lax.fori_loop(0, NB // 3, body, 0)
        wait_scatter((NB - 2) % 3)
        wait_scatter((NB - 1) % 3)
        plsc.subcore_barrier()

        pltpu.sync_copy(acc.at[pl.ds(s * RPT, RPT)],
                        msg_hbm.at[chunk, pl.ds(s * RPT, RPT)])
        plsc.subcore_barrier()

    run_chunk(0, (c * 2) * NP)
    run_chunk(1, NP)


def _sc_edge(src3, dst3, asrc, adst, xps):
    mesh = plsc.VectorSubcoreMesh(core_axis_name="c", subcore_axis_name="s",
                                  num_cores=2, num_subcores=NTILE)
    p1 = pl.kernel(
        _sc_pass1_body,
        compiler_params=pltpu.CompilerParams(needs_layout_passes=False),
        out_type=[
            jax.ShapeDtypeStruct((H, NTILE, NB, KB), jnp.float32),
            jax.ShapeDtypeStruct((H, NTILE, NP), jnp.float32),
        ],
        mesh=mesh,
        scratch_types=[
            pltpu.VMEM((NB, KB), jnp.int32),
            pltpu.VMEM((NB, KB), jnp.int32),
            pltpu.VMEM((NB, KB), jnp.float32),
            pltpu.VMEM((NP,), jnp.float32),
            pltpu.VMEM((NP,), jnp.float32),
            pltpu.VMEM((NP,), jnp.float32),
        ],
    )
    w, den = p1(src3, dst3, asrc, adst)

    p2 = pl.kernel(
        _sc_pass2_body,
        compiler_params=pltpu.CompilerParams(needs_layout_passes=False),
        out_type=[
            jax.ShapeDtypeStruct((4, NP, 128), jnp.float32),
        ],
        mesh=mesh,
        scratch_types=[
            pltpu.VMEM((NB, KB), jnp.int32),
            pltpu.VMEM((3, KB), jnp.int32),
            pltpu.VMEM((3, KB), jnp.float32),
            pltpu.VMEM((KB, 128), jnp.float32),
            pltpu.VMEM((KB, 128), jnp.float32),
            pltpu.VMEM((KB, 128), jnp.float32),
            pltpu.VMEM_SHARED((ACCR, 128), jnp.float32),
        ] + [pltpu.SemaphoreType.DMA] * 9,
    )
    (msg,) = p2(src3, dst3, w, xps.reshape(4 * NP, 128))
    return msg, den


def _mlp_body(msg0_ref, msg1_ref, msg2_ref, msg3_ref, den_ref,
              asrc_ref, adst_ref, xp0_ref, xp1_ref, xp2_ref, xp3_ref,
              bconv_ref, wa_ref, ba_ref, ga_ref, bta_ref,
              w1_ref, b1_ref, g1_ref, bt1_ref,
              w2_ref, b2_ref, g2_ref, bt2_ref,
              w3_ref, b3_ref, p_ref, sq_ref):
    def ln(v, g, b):
        mu = jnp.mean(v, axis=-1, keepdims=True)
        var = jnp.mean((v - mu) ** 2, axis=-1, keepdims=True)
        return (v - mu) * lax.rsqrt(var + 1e-5) * g + b

    wself = jnp.exp(_leaky(asrc_ref[...] + adst_ref[...]))
    den = jnp.sum(den_ref[...], axis=1) + wself + 1e-16
    inv0 = (1.0 / den[0])[:, None]
    inv1 = (1.0 / den[1])[:, None]
    ws0 = wself[0][:, None]
    ws1 = wself[1][:, None]
    h0 = jnp.concatenate([msg0_ref[0], msg1_ref[0]], axis=1)
    h1 = jnp.concatenate([msg2_ref[0], msg3_ref[0]], axis=1)
    xp0 = jnp.concatenate([xp0_ref[0], xp1_ref[0]], axis=1)
    xp1 = jnp.concatenate([xp2_ref[0], xp3_ref[0]], axis=1)
    h = jnp.concatenate([(h0 + ws0 * xp0) * inv0,
                         (h1 + ws1 * xp1) * inv1], axis=1)
    h = jnp.maximum(h + bconv_ref[...], 0.0)
    h = jnp.dot(h, wa_ref[...], preferred_element_type=jnp.float32) + ba_ref[...]
    h = ln(h, ga_ref[...], bta_ref[...])
    h = jnp.maximum(h, 0.0)
    h = jnp.dot(h, w1_ref[...], preferred_element_type=jnp.float32) + b1_ref[...]
    h = ln(h, g1_ref[...], bt1_ref[...])
    h = jnp.tanh(jnp.maximum(h, 0.0))
    h = jnp.dot(h, w2_ref[...], preferred_element_type=jnp.float32) + b2_ref[...]
    h = ln(h, g2_ref[...], bt2_ref[...])
    h = jnp.maximum(h, 0.0)
    p = jnp.dot(h, w3_ref[...], preferred_element_type=jnp.float32) + b3_ref[...]
    p_ref[...] = p
    sq_ref[0, :] = jnp.sum(p * p, axis=1)


def _mlp(msg, den, asrc, adst, xps, b_conv, Wa, ba, ga, bta,
         W1, b1, g1, bt1, W2, b2, g2, bt2, W3, b3):
    full = lambda r, c: pl.BlockSpec((r, c), lambda i: (0, 0))
    row = lambda c: pl.BlockSpec((1, c), lambda i: (0, 0))
    chunk = lambda cc: pl.BlockSpec((1, ROW_BLK, 128), lambda i, cc=cc: (cc, i, 0))
    hblk = pl.BlockSpec((H, ROW_BLK), lambda i: (0, i))
    return pl.pallas_call(
        _mlp_body,
        grid=(NP // ROW_BLK,),
        in_specs=[
            chunk(0), chunk(1), chunk(2), chunk(3),
            pl.BlockSpec((H, NTILE, ROW_BLK), lambda i: (0, 0, i)),
            hblk, hblk,
            chunk(0), chunk(1), chunk(2), chunk(3),
            row(H * C), full(H * C, 256), row(256), row(256), row(256),
            full(256, 128), row(128), row(128), row(128),
            full(128, 64), row(64), row(64), row(64),
            full(64, 3), row(3),
        ],
        out_specs=[pl.BlockSpec((ROW_BLK, 3), lambda i: (i, 0)),
                   pl.BlockSpec((1, ROW_BLK), lambda i: (0, i))],
        out_shape=[jax.ShapeDtypeStruct((N, 3), jnp.float32),
                   jax.ShapeDtypeStruct((1, NP), jnp.float32)],
    )(msg, msg, msg, msg, den, asrc, adst, xps, xps, xps, xps,
      b_conv.reshape(1, -1), Wa, ba.reshape(1, -1), ga.reshape(1, -1),
      bta.reshape(1, -1), W1, b1.reshape(1, -1), g1.reshape(1, -1),
      bt1.reshape(1, -1), W2, b2.reshape(1, -1), g2.reshape(1, -1),
      bt2.reshape(1, -1), W3, b3.reshape(1, -1))


CD_RB = 1024
CD_CB = 2048


def _cdist_body(pi_ref, pj_ref, sqj_ref, out_ref):
    pi = pi_ref[...]
    pj = pj_ref[...]
    dots = lax.dot_general(pi, pj, (((1,), (1,)), ((), ())),
                           preferred_element_type=jnp.float32)
    sq_i = jnp.sum(pi * pi, axis=1, keepdims=True)
    d2 = (sq_i + sqj_ref[...]) - 2.0 * dots
    out_ref[...] = jnp.sqrt(jnp.maximum(d2, 0.0))


def _cdist(p, sq):
    grid = (pl.cdiv(N, CD_RB), pl.cdiv(N, CD_CB))
    return pl.pallas_call(
        _cdist_body,
        grid=grid,
        in_specs=[
            pl.BlockSpec((CD_RB, 3), lambda i, j: (i, 0)),
            pl.BlockSpec((CD_CB, 3), lambda i, j: (j, 0)),
            pl.BlockSpec((1, CD_CB), lambda i, j: (0, j)),
        ],
        out_specs=pl.BlockSpec((CD_RB, CD_CB), lambda i, j: (i, j)),
        out_shape=jax.ShapeDtypeStruct((N, N), jnp.float32),
    )(p, p, sq)


def kernel(x, edge_index, W_conv, att_src, att_dst, b_conv, Wa, ba, ga, bta,
           W1, b1, g1, bt1, W2, b2, g2, bt2, W3, b3):
    xps, asrc, adst = _compute_xp(x, W_conv, att_src.reshape(1, H * C),
                                  att_dst.reshape(1, H * C))
    pad = EP - E
    src3 = jnp.concatenate(
        [edge_index[0], jnp.zeros((pad,), jnp.int32)]).reshape(NTILE, NB, KB)
    dst3 = jnp.concatenate(
        [edge_index[1], jnp.full((pad,), TRASH, jnp.int32)]).reshape(NTILE, NB, KB)
    msg, den = _sc_edge(src3, dst3, asrc, adst, xps)
    p, sq = _mlp(msg, den, asrc, adst, xps, b_conv, Wa, ba, ga, bta,
                 W1, b1, g1, bt1, W2, b2, g2, bt2, W3, b3)
    return _cdist(p, sq)


# speedup vs baseline: 1.7280x; 1.0233x over previous
import functools

import jax
import jax.numpy as jnp
from jax import lax
from jax.experimental import pallas as pl
from jax.experimental.pallas import tpu as pltpu
from jax.experimental.pallas import tpu_sc as plsc

N = 10000
E = 160000
D_IN = 512
H = 2
C = 256

NP = 10240
NTILE = 16
NB = 129
KB = 80
EP = NTILE * NB * KB
ACCR = 10112
RPT = ACCR // NTILE
ROW_BLK = 1024
TRASH = N


def _xp_body(x_ref, w_ref, attsrc_ref, attdst_ref, xps_ref, asrc_ref, adst_ref):
    xb = x_ref[...]
    xp = jnp.dot(xb, w_ref[...], preferred_element_type=jnp.float32)
    for cc in range(4):
        xps_ref[cc] = xp[:, cc * 128:(cc + 1) * 128]
    ws = xp * attsrc_ref[...]
    wd = xp * attdst_ref[...]
    asrc_ref[0, :] = jnp.sum(ws[:, :C], axis=1)
    asrc_ref[1, :] = jnp.sum(ws[:, C:], axis=1)
    adst_ref[0, :] = jnp.sum(wd[:, :C], axis=1)
    adst_ref[1, :] = jnp.sum(wd[:, C:], axis=1)


def _compute_xp(x, W_conv, att_src_flat, att_dst_flat):
    return pl.pallas_call(
        _xp_body,
        grid=(NP // ROW_BLK,),
        in_specs=[
            pl.BlockSpec((ROW_BLK, D_IN), lambda i: (i, 0)),
            pl.BlockSpec((D_IN, H * C), lambda i: (0, 0)),
            pl.BlockSpec((1, H * C), lambda i: (0, 0)),
            pl.BlockSpec((1, H * C), lambda i: (0, 0)),
        ],
        out_specs=[
            pl.BlockSpec((4, ROW_BLK, 128), lambda i: (0, i, 0)),
            pl.BlockSpec((H, ROW_BLK), lambda i: (0, i)),
            pl.BlockSpec((H, ROW_BLK), lambda i: (0, i)),
        ],
        out_shape=[
            jax.ShapeDtypeStruct((4, NP, 128), jnp.float32),
            jax.ShapeDtypeStruct((H, NP), jnp.float32),
            jax.ShapeDtypeStruct((H, NP), jnp.float32),
        ],
    )(x, W_conv, att_src_flat, att_dst_flat)


def _leaky(a):
    return jnp.maximum(a, 0.0) + 0.2 * jnp.minimum(a, 0.0)


def _sc_pass1_body(src_hbm, dst_hbm, asrc_hbm, adst_hbm,
                   w_hbm, den_hbm,
                   src_v, dst_v, w_v, asrc_v, adst_v, den_v):
    c = lax.axis_index("c")
    s = lax.axis_index("s")

    pltpu.sync_copy(src_hbm.at[s], src_v)
    pltpu.sync_copy(dst_hbm.at[s], dst_v)
    pltpu.sync_copy(asrc_hbm.at[c], asrc_v)
    pltpu.sync_copy(adst_hbm.at[c], adst_v)

    zero16 = jnp.zeros((16,), jnp.float32)

    def zden(i, _):
        den_v[pl.ds(i * 16, 16)] = zero16
        return 0
    lax.fori_loop(0, NP // 16, zden, 0)

    def p1(j, _):
        for g in range(KB // 16):
            sl = pl.ds(g * 16, 16)
            s16 = src_v[j, sl]
            d16 = dst_v[j, sl]
            a = plsc.load_gather(asrc_v, [s16]) + plsc.load_gather(adst_v, [d16])
            w = jnp.exp(_leaky(a))
            w_v[j, sl] = w
            plsc.addupdate_scatter(den_v, [d16], w)
        return 0
    lax.fori_loop(0, NB, p1, 0)

    pltpu.sync_copy(w_v, w_hbm.at[c, s])
    pltpu.sync_copy(den_v, den_hbm.at[c, s])


def _sc_pass2_body(src_hbm, dst_hbm, w_hbm, xps_hbm,
                   msg_hbm,
                   sidx_v, dstb, wb, rows0, rows1, rows2, acc,
                   ssem0, ssem1, ssem2, gsem0, gsem1, gsem2,
                   csem0, csem1, csem2):
    c = lax.axis_index("c")
    s = lax.axis_index("s")

    pltpu.sync_copy(src_hbm.at[s], sidx_v)

    zero16 = jnp.zeros((16,), jnp.float32)
    ssems = (ssem0, ssem1, ssem2)
    gsems = (gsem0, gsem1, gsem2)
    csems = (csem0, csem1, csem2)
    rows = (rows0, rows1, rows2)

    def gather(j, r):
        pltpu.async_copy(xps_hbm.at[sidx_v.at[j]], rows[r], gsems[r])

    def wait_gather(j, r):
        pltpu.make_async_copy(xps_hbm.at[sidx_v.at[j]], rows[r],
                              gsems[r]).wait()

    def stage(j, r):
        pltpu.async_copy(dst_hbm.at[s, j], dstb.at[r], ssems[r])
        pltpu.async_copy(w_hbm.at[c, s, j], wb.at[r], ssems[r])

    def wait_stage(j, r):
        pltpu.make_async_copy(dst_hbm.at[s, j], dstb.at[r], ssems[r]).wait()
        pltpu.make_async_copy(w_hbm.at[c, s, j], wb.at[r], ssems[r]).wait()

    def scatter(r2):
        pltpu.async_copy(rows[r2], acc.at[dstb.at[r2]], csems[r2], add=True)

    def wait_scatter(r2):
        pltpu.make_async_copy(rows[r2], acc.at[dstb.at[r2]],
                              csems[r2]).wait()

    def scale(r2):
        buf = rows[r2]

        def sgrp(g, _):
            base = pl.multiple_of(g * 16, 16)
            wv = wb[r2, pl.ds(base, 16)]
            for lane in range(16):
                wsplat = wv.at[jnp.full((16,), lane, jnp.int32)].get(
                    mode="promise_in_bounds")
                row = base + lane
                for q in range(8):
                    sl = pl.ds(q * 16, 16)
                    buf[row, sl] = buf[row, sl] * wsplat
            return 0
        lax.fori_loop(0, KB // 16, sgrp, 0)

    def run_chunk(cc, delta):
        def zrows(row, _):
            for g in range(8):
                rows0[row, pl.ds(g * 16, 16)] = zero16
            return 0
        lax.fori_loop(0, KB, zrows, 0)
        nfull, rem = RPT // KB, RPT % KB
        for k in range(nfull):
            pltpu.sync_copy(rows0, acc.at[pl.ds(s * RPT + k * KB, KB)])
        if rem:
            pltpu.sync_copy(rows0.at[pl.ds(0, rem)],
                            acc.at[pl.ds(s * RPT + nfull * KB, rem)])
        plsc.subcore_barrier()

        chunk = c * 2 + cc

        def mkidx(j, _):
            for g in range(KB // 16):
                sl = pl.ds(g * 16, 16)
                sidx_v[j, sl] = sidx_v[j, sl] + delta
            return 0
        lax.fori_loop(0, NB, mkidx, 0)

        gather(0, 0)
        stage(0, 0)

        def body(i, _):
            for t in range(3):
                j = i * 3 + t

                @pl.when(j >= 2)
                def _():
                    wait_scatter((t + 1) % 3)

                @pl.when(j + 1 < NB)
                def _():
                    gather(j + 1, (t + 1) % 3)
                    stage(j + 1, (t + 1) % 3)

                wait_gather(j, t)
                wait_stage(j, t)
                scale(t)
                scatter(t)
            return 0
        lax.fori_loop(0, NB // 3, body, 0)
        wait_scatter((NB - 2) % 3)
        wait_scatter((NB - 1) % 3)
        plsc.subcore_barrier()

        pltpu.sync_copy(acc.at[pl.ds(s * RPT, RPT)],
                        msg_hbm.at[chunk, pl.ds(s * RPT, RPT)])
        plsc.subcore_barrier()

    run_chunk(0, (c * 2) * NP)
    run_chunk(1, NP)


def _sc_edge(src3, dst3, asrc, adst, xps):
    mesh = plsc.VectorSubcoreMesh(core_axis_name="c", subcore_axis_name="s",
                                  num_cores=2, num_subcores=NTILE)
    p1 = pl.kernel(
        _sc_pass1_body,
        compiler_params=pltpu.CompilerParams(needs_layout_passes=False),
        out_type=[
            jax.ShapeDtypeStruct((H, NTILE, NB, KB), jnp.float32),
            jax.ShapeDtypeStruct((H, NTILE, NP), jnp.float32),
        ],
        mesh=mesh,
        scratch_types=[
            pltpu.VMEM((NB, KB), jnp.int32),
            pltpu.VMEM((NB, KB), jnp.int32),
            pltpu.VMEM((NB, KB), jnp.float32),
            pltpu.VMEM((NP,), jnp.float32),
            pltpu.VMEM((NP,), jnp.float32),
            pltpu.VMEM((NP,), jnp.float32),
        ],
    )
    w, den = p1(src3, dst3, asrc, adst)

    p2 = pl.kernel(
        _sc_pass2_body,
        compiler_params=pltpu.CompilerParams(needs_layout_passes=False),
        out_type=[
            jax.ShapeDtypeStruct((4, NP, 128), jnp.float32),
        ],
        mesh=mesh,
        scratch_types=[
            pltpu.VMEM((NB, KB), jnp.int32),
            pltpu.VMEM((3, KB), jnp.int32),
            pltpu.VMEM((3, KB), jnp.float32),
            pltpu.VMEM((KB, 128), jnp.float32),
            pltpu.VMEM((KB, 128), jnp.float32),
            pltpu.VMEM((KB, 128), jnp.float32),
            pltpu.VMEM_SHARED((ACCR, 128), jnp.float32),
        ] + [pltpu.SemaphoreType.DMA] * 9,
    )
    (msg,) = p2(src3, dst3, w, xps.reshape(4 * NP, 128))
    return msg, den


def _mlp_body(msg0_ref, msg1_ref, msg2_ref, msg3_ref, den_ref,
              asrc_ref, adst_ref, xp0_ref, xp1_ref, xp2_ref, xp3_ref,
              bconv_ref, wa_ref, ba_ref, ga_ref, bta_ref,
              w1_ref, b1_ref, g1_ref, bt1_ref,
              w2_ref, b2_ref, g2_ref, bt2_ref,
              w3_ref, b3_ref, p_ref, sq_ref):
    def ln(v, g, b):
        mu = jnp.mean(v, axis=-1, keepdims=True)
        var = jnp.mean((v - mu) ** 2, axis=-1, keepdims=True)
        return (v - mu) * lax.rsqrt(var + 1e-5) * g + b

    wself = jnp.exp(_leaky(asrc_ref[...] + adst_ref[...]))
    den = jnp.sum(den_ref[...], axis=1) + wself + 1e-16
    inv0 = (1.0 / den[0])[:, None]
    inv1 = (1.0 / den[1])[:, None]
    ws0 = wself[0][:, None]
    ws1 = wself[1][:, None]
    h0 = jnp.concatenate([msg0_ref[0], msg1_ref[0]], axis=1)
    h1 = jnp.concatenate([msg2_ref[0], msg3_ref[0]], axis=1)
    xp0 = jnp.concatenate([xp0_ref[0], xp1_ref[0]], axis=1)
    xp1 = jnp.concatenate([xp2_ref[0], xp3_ref[0]], axis=1)
    h = jnp.concatenate([(h0 + ws0 * xp0) * inv0,
                         (h1 + ws1 * xp1) * inv1], axis=1)
    h = jnp.maximum(h + bconv_ref[...], 0.0)
    h = jnp.dot(h, wa_ref[...], preferred_element_type=jnp.float32) + ba_ref[...]
    h = ln(h, ga_ref[...], bta_ref[...])
    h = jnp.maximum(h, 0.0)
    h = jnp.dot(h, w1_ref[...], preferred_element_type=jnp.float32) + b1_ref[...]
    h = ln(h, g1_ref[...], bt1_ref[...])
    h = jnp.tanh(jnp.maximum(h, 0.0))
    h = jnp.dot(h, w2_ref[...], preferred_element_type=jnp.float32) + b2_ref[...]
    h = ln(h, g2_ref[...], bt2_ref[...])
    h = jnp.maximum(h, 0.0)
    p = jnp.dot(h, w3_ref[...], preferred_element_type=jnp.float32) + b3_ref[...]
    p_ref[...] = p
    sq_ref[0, :] = jnp.sum(p * p, axis=1)


def _mlp(msg, den, asrc, adst, xps, b_conv, Wa, ba, ga, bta,
         W1, b1, g1, bt1, W2, b2, g2, bt2, W3, b3):
    full = lambda r, c: pl.BlockSpec((r, c), lambda i: (0, 0))
    row = lambda c: pl.BlockSpec((1, c), lambda i: (0, 0))
    chunk = lambda cc: pl.BlockSpec((1, ROW_BLK, 128), lambda i, cc=cc: (cc, i, 0))
    hblk = pl.BlockSpec((H, ROW_BLK), lambda i: (0, i))
    return pl.pallas_call(
        _mlp_body,
        grid=(NP // ROW_BLK,),
        in_specs=[
            chunk(0), chunk(1), chunk(2), chunk(3),
            pl.BlockSpec((H, NTILE, ROW_BLK), lambda i: (0, 0, i)),
            hblk, hblk,
            chunk(0), chunk(1), chunk(2), chunk(3),
            row(H * C), full(H * C, 256), row(256), row(256), row(256),
            full(256, 128), row(128), row(128), row(128),
            full(128, 64), row(64), row(64), row(64),
            full(64, 3), row(3),
        ],
        out_specs=[pl.BlockSpec((ROW_BLK, 3), lambda i: (i, 0)),
                   pl.BlockSpec((1, ROW_BLK), lambda i: (0, i))],
        out_shape=[jax.ShapeDtypeStruct((N, 3), jnp.float32),
                   jax.ShapeDtypeStruct((1, NP), jnp.float32)],
    )(msg, msg, msg, msg, den, asrc, adst, xps, xps, xps, xps,
      b_conv.reshape(1, -1), Wa, ba.reshape(1, -1), ga.reshape(1, -1),
      bta.reshape(1, -1), W1, b1.reshape(1, -1), g1.reshape(1, -1),
      bt1.reshape(1, -1), W2, b2.reshape(1, -1), g2.reshape(1, -1),
      bt2.reshape(1, -1), W3, b3.reshape(1, -1))


CD_RB = 1024
CD_CB = 2048


def _cdist_body(pi_ref, pj_ref, sqj_ref, out_ref):
    pi = pi_ref[...]
    pj = pj_ref[...]
    dots = lax.dot_general(pi, pj, (((1,), (1,)), ((), ())),
                           preferred_element_type=jnp.float32)
    sq_i = jnp.sum(pi * pi, axis=1, keepdims=True)
    d2 = (sq_i + sqj_ref[...]) - 2.0 * dots
    out_ref[...] = jnp.sqrt(jnp.maximum(d2, 0.0))


def _cdist(p, sq):
    grid = (pl.cdiv(N, CD_RB), pl.cdiv(N, CD_CB))
    return pl.pallas_call(
        _cdist_body,
        grid=grid,
        in_specs=[
            pl.BlockSpec((CD_RB, 3), lambda i, j: (i, 0)),
            pl.BlockSpec((CD_CB, 3), lambda i, j: (j, 0)),
            pl.BlockSpec((1, CD_CB), lambda i, j: (0, j)),
        ],
        out_specs=pl.BlockSpec((CD_RB, CD_CB), lambda i, j: (i, j)),
        out_shape=jax.ShapeDtypeStruct((N, N), jnp.float32),
    )(p, p, sq)


def kernel(x, edge_index, W_conv, att_src, att_dst, b_conv, Wa, ba, ga, bta,
           W1, b1, g1, bt1, W2, b2, g2, bt2, W3, b3):
    xps, asrc, adst = _compute_xp(x, W_conv, att_src.reshape(1, H * C),
                                  att_dst.reshape(1, H * C))
    pad = EP - E
    src3 = jnp.concatenate(
        [edge_index[0], jnp.zeros((pad,), jnp.int32)]).reshape(NTILE, NB, KB)
    dst3 = jnp.concatenate(
        [edge_index[1], jnp.full((pad,), TRASH, jnp.int32)]).reshape(NTILE, NB, KB)
    msg, den = _sc_edge(src3, dst3, asrc, adst, xps)
    p, sq = _mlp(msg, den, asrc, adst, xps, b_conv, Wa, ba, ga, bta,
                 W1, b1, g1, bt1, W2, b2, g2, bt2, W3, b3)
    return _cdist(p, sq)
